# Initial kernel scaffold; baseline (speedup 1.0000x reference)
import jax
import jax.numpy as jnp
from jax.experimental import pallas as pl


def kernel(x, edge_index, W1, b1, bn_gamma, bn_beta, W2, b2, ln_gamma, ln_beta, lin_W, lin_b):
    raise NotImplementedError("write your pallas kernel here")


# speedup vs baseline: 16.0972x; 16.0972x over previous
import functools

import jax
import jax.numpy as jnp
from jax import lax
from jax.experimental import pallas as pl
from jax.experimental.pallas import tpu as pltpu
from jax.experimental.pallas import tpu_sc as plsc

N = 10000
E = 320000
D = 128
NC = 2
NS = 16
NW = NC * NS
CH = 128
NCHUNK = 79
EPW = NCHUNK * CH
EPAD = EPW * NW
DH = D // 2
EPC = EPAD // NS
NCH2 = EPC // CH
NPAD = 10112
RPW = NPAD // NS

BLK = 1000
GRID = N // BLK

_mesh = plsc.VectorSubcoreMesh(core_axis_name="c", subcore_axis_name="s")


@functools.partial(
    pl.kernel,
    out_type=jax.ShapeDtypeStruct((NC, NPAD, 16), jnp.float32),
    mesh=_mesh,
    scratch_types=[
        pltpu.VMEM((NCHUNK, CH), jnp.int32),
        pltpu.VMEM((CH, 16), jnp.float32),
        pltpu.VMEM_SHARED((NPAD, 16), jnp.float32),
    ],
)
def _sc_degree(dst_hbm, zeros_hbm, out_hbm, dstv, onesv, acc_sh):
    c = lax.axis_index("c")
    s = lax.axis_index("s")
    wid = c * NS + s
    pltpu.sync_copy(dst_hbm.at[wid], dstv)

    @pl.loop(0, CH)
    def _(i):
        onesv[i, pl.ds(0, 16)] = jnp.ones((16,), jnp.float32)

    pltpu.sync_copy(zeros_hbm.at[pl.ds(s * RPW, RPW)],
                    acc_sh.at[pl.ds(s * RPW, RPW)])
    plsc.subcore_barrier()

    @pl.loop(0, NCHUNK)
    def _(j):
        pltpu.sync_copy(onesv, acc_sh.at[dstv.at[j]], add=True)

    plsc.subcore_barrier()
    pltpu.sync_copy(acc_sh.at[pl.ds(s * RPW, RPW)],
                    out_hbm.at[c, pl.ds(s * RPW, RPW)])


@functools.partial(
    pl.kernel,
    out_type=jax.ShapeDtypeStruct((NC, NPAD, DH), jnp.float32),
    mesh=_mesh,
    scratch_types=[
        pltpu.VMEM((NCH2, CH), jnp.int32),
        pltpu.VMEM((NCH2, CH), jnp.int32),
        pltpu.VMEM((CH, DH), jnp.float32),
        pltpu.VMEM((CH, DH), jnp.float32),
        pltpu.VMEM_SHARED((NPAD, DH), jnp.float32),
        pltpu.SemaphoreType.DMA,
        pltpu.SemaphoreType.DMA,
    ],
    compiler_params=pltpu.CompilerParams(use_tc_tiling_on_sc=False),
)
def _sc_segsum(g_hbm, src_hbm, dst_hbm, zeros_hbm, out_hbm,
               srcv, dstv, buf0, buf1, acc_sh, sem0, sem1):
    c = lax.axis_index("c")
    s = lax.axis_index("s")
    pltpu.sync_copy(src_hbm.at[s], srcv)
    pltpu.sync_copy(dst_hbm.at[s], dstv)
    pltpu.sync_copy(zeros_hbm.at[pl.ds(s * RPW, RPW)],
                    acc_sh.at[pl.ds(s * RPW, RPW)])
    plsc.subcore_barrier()
    gsrc = g_hbm.at[c]

    pltpu.async_copy(gsrc.at[srcv.at[0]], buf0, sem0)

    @pl.loop(0, NCH2, step=2)
    def _(j):
        pltpu.make_async_copy(gsrc.at[srcv.at[j]], buf0, sem0).wait()
        pltpu.async_copy(gsrc.at[srcv.at[j + 1]], buf1, sem1)
        pltpu.sync_copy(buf0, acc_sh.at[dstv.at[j]], add=True)
        pltpu.make_async_copy(gsrc.at[srcv.at[j + 1]], buf1, sem1).wait()

        @pl.when(j + 2 < NCH2)
        def _():
            pltpu.async_copy(gsrc.at[srcv.at[j + 2]], buf0, sem0)

        pltpu.sync_copy(buf1, acc_sh.at[dstv.at[j + 1]], add=True)

    plsc.subcore_barrier()
    pltpu.sync_copy(acc_sh.at[pl.ds(s * RPW, RPW)],
                    out_hbm.at[c, pl.ds(s * RPW, RPW)])


def _tc_pre1(degp_ref, x_ref, w1_ref, h1_ref, dinv_ref, g1_ref):
    deg = degp_ref[0] + degp_ref[1] + 1.0
    dinv = lax.rsqrt(deg)
    dinv_ref[...] = dinv
    h1 = jnp.dot(x_ref[...], w1_ref[...], preferred_element_type=jnp.float32)
    h1_ref[...] = h1
    g1 = h1 * dinv[:, 0:1]
    g1_ref[0] = g1[:, :DH]
    g1_ref[1] = g1[:, DH:]


def _tc_post1(p_ref, h1_ref, dinv_ref, b1_ref, out1_ref, stats_ref):
    d = dinv_ref[:, 0:1]
    p = jnp.concatenate([p_ref[0], p_ref[1]], axis=-1)
    acc = d * p + (d * d) * h1_ref[...] + b1_ref[...]
    out1_ref[...] = acc

    @pl.when(pl.program_id(0) == 0)
    def _():
        stats_ref[...] = jnp.zeros_like(stats_ref)

    stats_ref[0:1, :] += jnp.sum(acc, axis=0, keepdims=True)
    stats_ref[1:2, :] += jnp.sum(acc * acc, axis=0, keepdims=True)


def _tc_mid(out1_ref, stats_ref, bg_ref, bb_ref, w2_ref, dinv_ref,
            h2_ref, g2_ref):
    mean = stats_ref[0:1, :] * (1.0 / N)
    var = stats_ref[1:2, :] * (1.0 / N) - mean * mean
    x2 = (out1_ref[...] - mean) * lax.rsqrt(var + 1e-5) * bg_ref[...] + bb_ref[...]
    x2 = jnp.maximum(x2, 0.0)
    h2 = jnp.dot(x2, w2_ref[...], preferred_element_type=jnp.float32)
    h2_ref[...] = h2
    g2 = h2 * dinv_ref[:, 0:1]
    g2_ref[0] = g2[:, :DH]
    g2_ref[1] = g2[:, DH:]


def _tc_post2(p_ref, h2_ref, dinv_ref, b2_ref, lg_ref, lb_ref, lw_ref,
              lbias_ref, z_ref):
    d = dinv_ref[:, 0:1]
    p = jnp.concatenate([p_ref[0], p_ref[1]], axis=-1)
    acc = d * p + (d * d) * h2_ref[...] + b2_ref[...]
    mu = jnp.mean(acc, axis=-1, keepdims=True)
    v = jnp.mean(acc * acc, axis=-1, keepdims=True) - mu * mu
    y = (acc - mu) * lax.rsqrt(v + 1e-5) * lg_ref[...] + lb_ref[...]
    y = jnp.maximum(y, 0.0)
    z = lax.dot_general(y, lw_ref[...], (((1,), (1,)), ((), ())),
                        preferred_element_type=jnp.float32)
    z_ref[...] = z + lbias_ref[...]


def _row_spec(width):
    return pl.BlockSpec((BLK, width), lambda i: (i, 0))


def _full_spec(shape):
    return pl.BlockSpec(shape, lambda i: tuple(0 for _ in shape))


def kernel(x, edge_index, W1, b1, bn_gamma, bn_beta, W2, b2,
           ln_gamma, ln_beta, lin_W, lin_b):
    f32 = jnp.float32
    src = edge_index[0].astype(jnp.int32)
    dst = edge_index[1].astype(jnp.int32)
    pad = EPAD - E
    src_p = jnp.concatenate([src, jnp.zeros((pad,), jnp.int32)])
    dst_p = jnp.concatenate([dst, jnp.full((pad,), N, jnp.int32)])
    src_deg = src_p.reshape(NW, NCHUNK, CH)
    dst_deg = dst_p.reshape(NW, NCHUNK, CH)
    src_seg = src_p.reshape(NS, NCH2, CH)
    dst_seg = dst_p.reshape(NS, NCH2, CH)
    zeros16 = jnp.zeros((NPAD, 16), f32)
    zerosH = jnp.zeros((NPAD, DH), f32)
    b1r = b1.reshape(1, D)
    b2r = b2.reshape(1, D)
    bgr = bn_gamma.reshape(1, D)
    bbr = bn_beta.reshape(1, D)
    lgr = ln_gamma.reshape(1, D)
    lbr = ln_beta.reshape(1, D)
    lbias = lin_b.reshape(1, D)

    degp = _sc_degree(dst_deg, zeros16)

    gsplit_spec = pl.BlockSpec((NC, BLK, DH), lambda i: (0, i, 0))
    gsplit_shape = jax.ShapeDtypeStruct((NC, N, DH), f32)

    h1, dinv, g1 = pl.pallas_call(
        _tc_pre1,
        grid=(GRID,),
        in_specs=[pl.BlockSpec((NC, BLK, 16), lambda i: (0, i, 0)),
                  _row_spec(D), _full_spec((D, D))],
        out_specs=[_row_spec(D), _row_spec(16), gsplit_spec],
        out_shape=[jax.ShapeDtypeStruct((N, D), f32),
                   jax.ShapeDtypeStruct((N, 16), f32),
                   gsplit_shape],
    )(degp, x, W1)

    p1 = _sc_segsum(g1, src_seg, dst_seg, zerosH)

    out1, stats = pl.pallas_call(
        _tc_post1,
        grid=(GRID,),
        in_specs=[pl.BlockSpec((NC, BLK, DH), lambda i: (0, i, 0)),
                  _row_spec(D), _row_spec(16), _full_spec((1, D))],
        out_specs=[_row_spec(D), _full_spec((8, D))],
        out_shape=[jax.ShapeDtypeStruct((N, D), f32),
                   jax.ShapeDtypeStruct((8, D), f32)],
    )(p1, h1, dinv, b1r)

    h2, g2 = pl.pallas_call(
        _tc_mid,
        grid=(GRID,),
        in_specs=[_row_spec(D), _full_spec((8, D)), _full_spec((1, D)),
                  _full_spec((1, D)), _full_spec((D, D)), _row_spec(16)],
        out_specs=[_row_spec(D), gsplit_spec],
        out_shape=[jax.ShapeDtypeStruct((N, D), f32), gsplit_shape],
    )(out1, stats, bgr, bbr, W2, dinv)

    p2 = _sc_segsum(g2, src_seg, dst_seg, zerosH)

    z = pl.pallas_call(
        _tc_post2,
        grid=(GRID,),
        in_specs=[pl.BlockSpec((NC, BLK, DH), lambda i: (0, i, 0)),
                  _row_spec(D), _row_spec(16), _full_spec((1, D)),
                  _full_spec((1, D)), _full_spec((1, D)), _full_spec((D, D)),
                  _full_spec((1, D))],
        out_specs=_row_spec(D),
        out_shape=jax.ShapeDtypeStruct((N, D), f32),
    )(p2, h2, dinv, b2r, lgr, lbr, lin_W, lbias)

    return z
